# baseline (speedup 1.0000x reference)
import functools

import jax
import jax.numpy as jnp
from jax import lax
from jax.experimental import pallas as pl
from jax.experimental.pallas import tpu as pltpu
from jax.experimental.pallas import tpu_sc as plsc

D = 64
NC = 2
NS = 16
NW = NC * NS
CHUNK = 256
NBUF = 4
PREF = 2


@functools.partial(jax.jit, static_argnums=(3,))
def _gather_add(idx, embed_table, phoneme_table, n_rows):
    b_per_w = n_rows // NW
    n_chunks = b_per_w // CHUNK
    mesh = plsc.VectorSubcoreMesh(core_axis_name="c", subcore_axis_name="s")

    @functools.partial(
        pl.kernel,
        mesh=mesh,
        compiler_params=pltpu.CompilerParams(use_tc_tiling_on_sc=False),
        out_type=jax.ShapeDtypeStruct((n_rows, D), jnp.float32),
        scratch_types=[
            pltpu.VMEM((b_per_w,), jnp.int32),
            pltpu.VMEM((NBUF, CHUNK, D), jnp.float32),
            pltpu.SemaphoreType.DMA((NBUF,)),
            pltpu.SemaphoreType.DMA((NBUF,)),
            pltpu.SemaphoreType.DMA((NBUF,)),
        ],
    )
    def k(idx_hbm, embed_hbm, phon_hbm, out_hbm, idx_v, rows,
          esem, psem, wsem):
        wid = lax.axis_index("s") * NC + lax.axis_index("c")
        base = wid * b_per_w

        pltpu.sync_copy(idx_hbm.at[pl.ds(base, b_per_w)], idx_v)

        def isl(g):
            return idx_v.at[pl.ds(g * CHUNK, CHUNK)]

        def start_e(g, b):
            pltpu.async_copy(embed_hbm.at[isl(g)], rows.at[b], esem.at[b])

        def wait(table, b, sem):
            pltpu.make_async_copy(table.at[isl(0)], rows.at[b],
                                  sem.at[b]).wait()

        def wait_write(b):
            pltpu.make_async_copy(rows.at[b],
                                  out_hbm.at[pl.ds(base, CHUNK)],
                                  wsem.at[b]).wait()

        for h in range(PREF):
            start_e(h, h)

        def chunk_body(g, _):
            b = lax.rem(g, NBUF)

            wait(embed_hbm, b, esem)
            pltpu.async_copy(phon_hbm.at[isl(g)], rows.at[b], psem.at[b],
                             add=True)

            h = g + PREF

            @pl.when(h < n_chunks)
            def _():
                hb = lax.rem(h, NBUF)

                @pl.when(h >= NBUF)
                def _():
                    wait_write(hb)

                start_e(h, hb)

            wait(phon_hbm, b, psem)
            pltpu.async_copy(rows.at[b],
                             out_hbm.at[pl.ds(base + g * CHUNK, CHUNK)],
                             wsem.at[b])
            return ()

        lax.fori_loop(0, n_chunks, chunk_body, ())
        for b in range(NBUF):
            wait_write(b)

    return k(idx, embed_table, phoneme_table)


def kernel(x, embed_table, phoneme_table):
    n_rows = x.shape[0] * x.shape[1]
    idx = x.reshape(n_rows).astype(jnp.int32)
    split = (n_rows * 3 // 5) // 8192 * 8192
    out_sc = _gather_add(idx[:split], embed_table, phoneme_table, split)
    idx_tc = idx[split:]
    out_tc = (jnp.take(embed_table, idx_tc, axis=0)
              + jnp.take(phoneme_table, idx_tc, axis=0))
    out = jnp.concatenate([out_sc, out_tc], axis=0)
    return out.reshape(x.shape[0], x.shape[1], D)


# speedup vs baseline: 1.0235x; 1.0235x over previous
import functools

import jax
import jax.numpy as jnp
from jax import lax
from jax.experimental import pallas as pl
from jax.experimental.pallas import tpu as pltpu
from jax.experimental.pallas import tpu_sc as plsc

D = 64
NC = 2
NS = 16
NW = NC * NS
CHUNK = 512
NBUF = 2
PREF = 1


@functools.partial(jax.jit, static_argnums=(3,))
def _gather_add(idx, embed_table, phoneme_table, n_rows):
    b_per_w = n_rows // NW
    n_chunks = b_per_w // CHUNK
    mesh = plsc.VectorSubcoreMesh(core_axis_name="c", subcore_axis_name="s")

    @functools.partial(
        pl.kernel,
        mesh=mesh,
        compiler_params=pltpu.CompilerParams(use_tc_tiling_on_sc=False),
        out_type=jax.ShapeDtypeStruct((n_rows, D), jnp.bfloat16),
        scratch_types=[
            pltpu.VMEM((b_per_w,), jnp.int32),
            pltpu.VMEM((NBUF, CHUNK, D), jnp.bfloat16),
            pltpu.SemaphoreType.DMA((NBUF,)),
            pltpu.SemaphoreType.DMA((NBUF,)),
            pltpu.SemaphoreType.DMA((NBUF,)),
        ],
    )
    def k(idx_hbm, embed_hbm, phon_hbm, out_hbm, idx_v, rows,
          esem, psem, wsem):
        wid = lax.axis_index("s") * NC + lax.axis_index("c")
        base = wid * b_per_w

        pltpu.sync_copy(idx_hbm.at[pl.ds(base, b_per_w)], idx_v)

        def isl(g):
            return idx_v.at[pl.ds(g * CHUNK, CHUNK)]

        def start_e(g, b):
            pltpu.async_copy(embed_hbm.at[isl(g)], rows.at[b], esem.at[b])

        def wait(table, b, sem):
            pltpu.make_async_copy(table.at[isl(0)], rows.at[b],
                                  sem.at[b]).wait()

        def wait_write(b):
            pltpu.make_async_copy(rows.at[b],
                                  out_hbm.at[pl.ds(base, CHUNK)],
                                  wsem.at[b]).wait()

        for h in range(PREF):
            start_e(h, h)

        def chunk_body(g, _):
            b = lax.rem(g, NBUF)

            wait(embed_hbm, b, esem)
            pltpu.async_copy(phon_hbm.at[isl(g)], rows.at[b], psem.at[b],
                             add=True)

            h = g + PREF

            @pl.when(h < n_chunks)
            def _():
                hb = lax.rem(h, NBUF)

                @pl.when(h >= NBUF)
                def _():
                    wait_write(hb)

                start_e(h, hb)

            wait(phon_hbm, b, psem)
            pltpu.async_copy(rows.at[b],
                             out_hbm.at[pl.ds(base + g * CHUNK, CHUNK)],
                             wsem.at[b])
            return ()

        lax.fori_loop(0, n_chunks, chunk_body, ())
        for b in range(NBUF):
            wait_write(b)

    return k(idx, embed_table, phoneme_table)


def kernel(x, embed_table, phoneme_table):
    n_rows = x.shape[0] * x.shape[1]
    idx = x.reshape(n_rows).astype(jnp.int32)
    eb = embed_table.astype(jnp.bfloat16)
    pb = phoneme_table.astype(jnp.bfloat16)
    out = _gather_add(idx, eb, pb, n_rows)
    return out.astype(jnp.float32).reshape(x.shape[0], x.shape[1], D)


# speedup vs baseline: 1.4762x; 1.4422x over previous
import functools

import jax
import jax.numpy as jnp
from jax import lax
from jax.experimental import pallas as pl
from jax.experimental.pallas import tpu as pltpu
from jax.experimental.pallas import tpu_sc as plsc

D = 64
NC = 2
NS = 16
NW = NC * NS
CHUNK = 512
NBUF = 2
PREF = 1


@functools.partial(jax.jit, static_argnums=(3,))
def _gather_add(idx, embed_table, phoneme_table, n_rows):
    b_per_w = n_rows // NW
    n_chunks = b_per_w // CHUNK
    mesh = plsc.VectorSubcoreMesh(core_axis_name="c", subcore_axis_name="s")

    @functools.partial(
        pl.kernel,
        mesh=mesh,
        compiler_params=pltpu.CompilerParams(use_tc_tiling_on_sc=False),
        out_type=jax.ShapeDtypeStruct((n_rows, D), jnp.float32),
        scratch_types=[
            pltpu.VMEM((b_per_w,), jnp.int32),
            pltpu.VMEM((NBUF, CHUNK, D), jnp.float32),
            pltpu.SemaphoreType.DMA((NBUF,)),
            pltpu.SemaphoreType.DMA((NBUF,)),
            pltpu.SemaphoreType.DMA((NBUF,)),
        ],
    )
    def k(idx_hbm, embed_hbm, phon_hbm, out_hbm, idx_v, rows,
          esem, psem, wsem):
        wid = lax.axis_index("s") * NC + lax.axis_index("c")
        base = wid * b_per_w

        pltpu.sync_copy(idx_hbm.at[pl.ds(base, b_per_w)], idx_v)

        def isl(g):
            return idx_v.at[pl.ds(g * CHUNK, CHUNK)]

        def start_e(g, b):
            pltpu.async_copy(embed_hbm.at[isl(g)], rows.at[b], esem.at[b])

        def wait(table, b, sem):
            pltpu.make_async_copy(table.at[isl(0)], rows.at[b],
                                  sem.at[b]).wait()

        def wait_write(b):
            pltpu.make_async_copy(rows.at[b],
                                  out_hbm.at[pl.ds(base, CHUNK)],
                                  wsem.at[b]).wait()

        for h in range(PREF):
            start_e(h, h)

        def chunk_body(g, _):
            b = lax.rem(g, NBUF)

            wait(embed_hbm, b, esem)
            pltpu.async_copy(phon_hbm.at[isl(g)], rows.at[b], psem.at[b],
                             add=True)

            h = g + PREF

            @pl.when(h < n_chunks)
            def _():
                hb = lax.rem(h, NBUF)

                @pl.when(h >= NBUF)
                def _():
                    wait_write(hb)

                start_e(h, hb)

            wait(phon_hbm, b, psem)
            pltpu.async_copy(rows.at[b],
                             out_hbm.at[pl.ds(base + g * CHUNK, CHUNK)],
                             wsem.at[b])
            return ()

        lax.fori_loop(0, n_chunks, chunk_body, ())
        for b in range(NBUF):
            wait_write(b)

    return k(idx, embed_table, phoneme_table)


def kernel(x, embed_table, phoneme_table):
    n_rows = x.shape[0] * x.shape[1]
    idx = x.reshape(n_rows).astype(jnp.int32)
    out = _gather_add(idx, embed_table, phoneme_table, n_rows)
    return out.reshape(x.shape[0], x.shape[1], D)
